# baseline (speedup 1.0000x reference)
import functools

import jax
import jax.numpy as jnp
from jax import lax
from jax.experimental import pallas as pl
from jax.experimental.pallas import tpu as pltpu
from jax.experimental.pallas import tpu_sc as plsc

NE = 8
DM = 1024
DF = 512
T = 2048
BM2 = 256
NBLK = 24
P = NBLK * BM2
NSC = 2
NTILE = 16
NW = NSC * NTILE
TPW = T // NW


def _route_kernel(x_ref, protoT_ref, logits_ref, meta_ref, blk_ref):
    x = x_ref[...]
    xsq = jnp.sum(x * x, axis=1, keepdims=True)
    xn = x / jnp.maximum(jnp.sqrt(xsq), 1e-12)
    pT = protoT_ref[...]
    psq = jnp.sum(pT * pT, axis=0, keepdims=True)
    pn = pT / jnp.maximum(jnp.sqrt(psq), 1e-12)
    logits = lax.dot_general(
        xn.astype(jnp.bfloat16), pn.astype(jnp.bfloat16),
        (((1,), (0,)), ((), ())), preferred_element_type=jnp.float32)
    logits_ref[...] = logits

    m = jnp.max(logits, axis=1, keepdims=True)
    ex = jnp.exp(logits - m)
    probs = ex / jnp.sum(ex, axis=1, keepdims=True)
    p1 = jnp.max(probs, axis=1, keepdims=True)
    masked = jnp.where(probs >= p1, -jnp.inf, probs)
    p2 = jnp.max(masked, axis=1, keepdims=True)
    sel = probs >= p2

    mask_i = sel.astype(jnp.int32)
    c = mask_i
    s = 1
    while s < T:
        c = c + jnp.concatenate(
            [jnp.zeros((s, NE), jnp.int32), c[:T - s]], axis=0)
        s *= 2
    excl = c - mask_i

    lane1 = lax.broadcasted_iota(jnp.int32, (1, NE), 1)
    offs_row = jnp.zeros((1, NE), jnp.int32)
    blk = jnp.zeros((1, NBLK), jnp.int32)
    bstart = lax.broadcasted_iota(jnp.int32, (1, NBLK), 1) * BM2
    off_e = jnp.zeros((), jnp.int32)
    for e in range(NE):
        cnt_e = jnp.sum(mask_i[:, e:e + 1])
        cp_e = ((cnt_e + BM2 - 1) >> 8) << 8
        offs_row = offs_row + jnp.where(lane1 == e, off_e, 0)
        off_e = off_e + cp_e
        blk = blk + (bstart >= off_e).astype(jnp.int32)
    blk_ref[...] = blk

    dest = offs_row + excl
    one1 = probs >= p1
    one2 = sel & (~one1)
    d0 = jnp.sum(jnp.where(one1, dest, 0), axis=1, keepdims=True)
    d1 = jnp.sum(jnp.where(one2, dest, 0), axis=1, keepdims=True)
    w0 = lax.bitcast_convert_type(p1, jnp.int32)
    w1 = lax.bitcast_convert_type(p2, jnp.int32)
    meta = jnp.concatenate([d0, d1, w0, w1], axis=1)
    meta_ref[...] = meta


def _route(x, protoT):
    return pl.pallas_call(
        _route_kernel,
        in_specs=[
            pl.BlockSpec((T, DM), lambda: (0, 0)),
            pl.BlockSpec((DM, NE), lambda: (0, 0)),
        ],
        out_specs=[
            pl.BlockSpec((T, NE), lambda: (0, 0)),
            pl.BlockSpec((T, 4), lambda: (0, 0)),
            pl.BlockSpec((1, NBLK), lambda: (0, 0)),
        ],
        out_shape=[
            jax.ShapeDtypeStruct((T, NE), jnp.float32),
            jax.ShapeDtypeStruct((T, 4), jnp.int32),
            jax.ShapeDtypeStruct((1, NBLK), jnp.int32),
        ],
    )(x, protoT)


def _dispatch_kernel(x_hbm, meta_hbm, xs_hbm, wsort_hbm,
                     meta_v, rows_v, sem):
    c = lax.axis_index("c")
    s = lax.axis_index("s")
    wid = s * NSC + c
    base = wid * TPW
    pltpu.sync_copy(meta_hbm.at[wid], meta_v)
    pltpu.sync_copy(x_hbm.at[pl.ds(base, TPW)], rows_v)
    cps = [
        pltpu.make_async_copy(rows_v, xs_hbm.at[meta_v.at[0]], sem),
        pltpu.make_async_copy(rows_v, xs_hbm.at[meta_v.at[1]], sem),
        pltpu.make_async_copy(meta_v.at[2], wsort_hbm.at[meta_v.at[0]], sem),
        pltpu.make_async_copy(meta_v.at[3], wsort_hbm.at[meta_v.at[1]], sem),
    ]
    for cp in cps:
        cp.start()
    for cp in cps:
        cp.wait()


def _dispatch(x, meta_rs):
    mesh = plsc.VectorSubcoreMesh(core_axis_name="c", subcore_axis_name="s")
    f = pl.kernel(
        _dispatch_kernel,
        out_type=[
            jax.ShapeDtypeStruct((P, DM // 2), jnp.int32),
            jax.ShapeDtypeStruct((P,), jnp.int32),
        ],
        mesh=mesh,
        scratch_types=[
            pltpu.VMEM((4, TPW), jnp.int32),
            pltpu.VMEM((TPW, DM // 2), jnp.int32),
            pltpu.SemaphoreType.DMA,
        ],
    )
    return f(x, meta_rs)


def _mm(a, b):
    return lax.dot_general(a, b, (((1,), (0,)), ((), ())),
                           preferred_element_type=jnp.float32)


def _gemm_kernel(blk_ref, xs_ref, ws_ref, wge_ref, wgo_ref, wue_ref,
                 wuo_ref, wd_ref, y_ref):
    e = blk_ref[0, pl.program_id(0)]

    @pl.when(e < NE)
    def _body():
        xi = xs_ref[...]
        xe = lax.bitcast_convert_type(
            xi << 16, jnp.float32).astype(jnp.bfloat16)
        xo = lax.bitcast_convert_type(
            xi & jnp.int32(-65536), jnp.float32).astype(jnp.bfloat16)
        g = _mm(xe, wge_ref[e]) + _mm(xo, wgo_ref[e])
        u = _mm(xe, wue_ref[e]) + _mm(xo, wuo_ref[e])
        h = (g / (1.0 + jnp.exp(-g))) * u
        hb = (h * ws_ref[...]).astype(jnp.bfloat16)
        y_ref[...] = _mm(hb, wd_ref[e])


def _gemm(blk, xs_bits, wcol, wge, wgo, wue, wuo, wd):
    return pl.pallas_call(
        _gemm_kernel,
        grid=(NBLK,),
        in_specs=[
            pl.BlockSpec(memory_space=pltpu.MemorySpace.SMEM),
            pl.BlockSpec((BM2, DM // 2), lambda i: (i, 0)),
            pl.BlockSpec((BM2, 1), lambda i: (i, 0)),
            pl.BlockSpec((NE, DM // 2, DF), lambda i: (0, 0, 0)),
            pl.BlockSpec((NE, DM // 2, DF), lambda i: (0, 0, 0)),
            pl.BlockSpec((NE, DM // 2, DF), lambda i: (0, 0, 0)),
            pl.BlockSpec((NE, DM // 2, DF), lambda i: (0, 0, 0)),
            pl.BlockSpec((NE, DF, DM), lambda i: (0, 0, 0)),
        ],
        out_specs=pl.BlockSpec((BM2, DM), lambda i: (i, 0)),
        out_shape=jax.ShapeDtypeStruct((P, DM), jnp.float32),
    )(blk, xs_bits, wcol, wge, wgo, wue, wuo, wd)


def _combine_kernel(y_hbm, d0_hbm, d1_hbm, out_hbm,
                    idx0_v, idx1_v, buf0_v, buf1_v, sem):
    c = lax.axis_index("c")
    s = lax.axis_index("s")
    half = TPW // 2
    for h in range(2):
        row = (c * NTILE + s) * 2 + h
        pltpu.sync_copy(d0_hbm.at[row], idx0_v)
        pltpu.sync_copy(d1_hbm.at[row], idx1_v)
        cp0 = pltpu.make_async_copy(y_hbm.at[idx0_v], buf0_v, sem)
        cp0.start()
        cp1 = pltpu.make_async_copy(y_hbm.at[idx1_v], buf1_v, sem)
        cp1.start()
        cp0.wait()
        cp1.wait()

        def _add_row(r, _):
            for j in range(DM // 16):
                sl = pl.ds(j * 16, 16)
                buf0_v[r, sl] = buf0_v[r, sl] + buf1_v[r, sl]
            return 0

        lax.fori_loop(0, half, _add_row, 0)
        base = (c * NTILE + s) * TPW + h * half
        pltpu.sync_copy(buf0_v, out_hbm.at[pl.ds(base, half)])


def _combine(y, d0_rs, d1_rs):
    mesh = plsc.VectorSubcoreMesh(core_axis_name="c", subcore_axis_name="s")
    half = TPW // 2
    f = pl.kernel(
        _combine_kernel,
        out_type=jax.ShapeDtypeStruct((T, DM), jnp.float32),
        mesh=mesh,
        scratch_types=[
            pltpu.VMEM((half,), jnp.int32),
            pltpu.VMEM((half,), jnp.int32),
            pltpu.VMEM((half, DM), jnp.float32),
            pltpu.VMEM((half, DM), jnp.float32),
            pltpu.SemaphoreType.DMA,
        ],
    )
    return f(y, d0_rs, d1_rs)


@jax.jit
def kernel(hidden_states, proto, w_gate, w_up, w_down):
    B, S, D = hidden_states.shape
    x = hidden_states.reshape(T, D)
    protoT = proto.T
    wg = w_gate.astype(jnp.bfloat16)
    wu = w_up.astype(jnp.bfloat16)
    wd = w_down.astype(jnp.bfloat16)
    wge, wgo = wg[:, 0::2, :], wg[:, 1::2, :]
    wue, wuo = wu[:, 0::2, :], wu[:, 1::2, :]

    logits, meta, blk = _route(x, protoT)

    meta_rs = meta.reshape(NW, TPW, 4).transpose(0, 2, 1)
    x_bits = lax.bitcast_convert_type(
        x.astype(jnp.bfloat16).reshape(T, DM // 2, 2), jnp.int32)
    xs_bits, wsort = _dispatch(x_bits, meta_rs)

    wcol = lax.bitcast_convert_type(wsort, jnp.float32).reshape(P, 1)
    y = _gemm(blk, xs_bits, wcol, wge, wgo, wue, wuo, wd)

    d0_rs = meta[:, 0].reshape(64, 32)
    d1_rs = meta[:, 1].reshape(64, 32)
    out = _combine(y, d0_rs, d1_rs)
    return out.reshape(B, S, D), logits


# speedup vs baseline: 2.2854x; 2.2854x over previous
import functools

import jax
import jax.numpy as jnp
from jax import lax
from jax.experimental import pallas as pl
from jax.experimental.pallas import tpu as pltpu
from jax.experimental.pallas import tpu_sc as plsc

NE = 8
DM = 1024
DF = 512
T = 2048
BM2 = 256
NBLK = 24
P = NBLK * BM2
NSC = 2
NTILE = 16
NW = NSC * NTILE
TPW = T // NW


def _route_kernel(x_ref, protoT_ref, logits_ref, meta_ref, blk_ref,
                  xbits_ref):
    x = x_ref[...]
    xsq = jnp.sum(x * x, axis=1, keepdims=True)
    xn = x / jnp.maximum(jnp.sqrt(xsq), 1e-12)
    pT = protoT_ref[...]
    psq = jnp.sum(pT * pT, axis=0, keepdims=True)
    pn = pT / jnp.maximum(jnp.sqrt(psq), 1e-12)
    logits = lax.dot_general(
        xn.astype(jnp.bfloat16), pn.astype(jnp.bfloat16),
        (((1,), (0,)), ((), ())), preferred_element_type=jnp.float32)
    logits_ref[...] = logits

    m = jnp.max(logits, axis=1, keepdims=True)
    ex = jnp.exp(logits - m)
    probs = ex / jnp.sum(ex, axis=1, keepdims=True)
    p1 = jnp.max(probs, axis=1, keepdims=True)
    masked = jnp.where(probs >= p1, -jnp.inf, probs)
    p2 = jnp.max(masked, axis=1, keepdims=True)
    sel = probs >= p2

    mask_i = sel.astype(jnp.int32)
    c = mask_i
    s = 1
    while s < T:
        c = c + jnp.concatenate(
            [jnp.zeros((s, NE), jnp.int32), c[:T - s]], axis=0)
        s *= 2
    excl = c - mask_i

    lane1 = lax.broadcasted_iota(jnp.int32, (1, NE), 1)
    offs_row = jnp.zeros((1, NE), jnp.int32)
    blk = jnp.zeros((1, NBLK), jnp.int32)
    bstart = lax.broadcasted_iota(jnp.int32, (1, NBLK), 1) * BM2
    off_e = jnp.zeros((), jnp.int32)
    for e in range(NE):
        cnt_e = jnp.sum(mask_i[:, e:e + 1])
        cp_e = ((cnt_e + BM2 - 1) >> 8) << 8
        offs_row = offs_row + jnp.where(lane1 == e, off_e, 0)
        off_e = off_e + cp_e
        blk = blk + (bstart >= off_e).astype(jnp.int32)
    blk_ref[...] = blk

    dest = offs_row + excl
    one1 = probs >= p1
    one2 = sel & (~one1)
    d0 = jnp.sum(jnp.where(one1, dest, 0), axis=1, keepdims=True)
    d1 = jnp.sum(jnp.where(one2, dest, 0), axis=1, keepdims=True)
    w0 = lax.bitcast_convert_type(p1, jnp.int32)
    w1 = lax.bitcast_convert_type(p2, jnp.int32)
    meta = jnp.concatenate([d0, d1, w0, w1], axis=1)
    meta_ref[...] = meta

    lo = lax.shift_right_logical(
        lax.bitcast_convert_type(
            x[:, :DM // 2].astype(jnp.bfloat16).astype(jnp.float32),
            jnp.int32), 16)
    hi = lax.bitcast_convert_type(
        x[:, DM // 2:].astype(jnp.bfloat16).astype(jnp.float32),
        jnp.int32) & jnp.int32(-65536)
    xbits_ref[...] = lo | hi


def _route(x, protoT):
    return pl.pallas_call(
        _route_kernel,
        in_specs=[
            pl.BlockSpec((T, DM), lambda: (0, 0)),
            pl.BlockSpec((DM, NE), lambda: (0, 0)),
        ],
        out_specs=[
            pl.BlockSpec((T, NE), lambda: (0, 0)),
            pl.BlockSpec((T, 4), lambda: (0, 0)),
            pl.BlockSpec((1, NBLK), lambda: (0, 0)),
            pl.BlockSpec((T, DM // 2), lambda: (0, 0)),
        ],
        out_shape=[
            jax.ShapeDtypeStruct((T, NE), jnp.float32),
            jax.ShapeDtypeStruct((T, 4), jnp.int32),
            jax.ShapeDtypeStruct((1, NBLK), jnp.int32),
            jax.ShapeDtypeStruct((T, DM // 2), jnp.int32),
        ],
    )(x, protoT)


def _dispatch_kernel(x_hbm, meta_hbm, xs_hbm, wsort_hbm,
                     meta_v, rows_v, sem):
    c = lax.axis_index("c")
    s = lax.axis_index("s")
    wid = s * NSC + c
    base = wid * TPW
    pltpu.sync_copy(meta_hbm.at[wid], meta_v)
    pltpu.sync_copy(x_hbm.at[pl.ds(base, TPW)], rows_v)
    cps = [
        pltpu.make_async_copy(rows_v, xs_hbm.at[meta_v.at[0]], sem),
        pltpu.make_async_copy(rows_v, xs_hbm.at[meta_v.at[1]], sem),
        pltpu.make_async_copy(meta_v.at[2], wsort_hbm.at[meta_v.at[0]], sem),
        pltpu.make_async_copy(meta_v.at[3], wsort_hbm.at[meta_v.at[1]], sem),
    ]
    for cp in cps:
        cp.start()
    for cp in cps:
        cp.wait()


def _dispatch(x, meta_rs):
    mesh = plsc.VectorSubcoreMesh(core_axis_name="c", subcore_axis_name="s")
    f = pl.kernel(
        _dispatch_kernel,
        out_type=[
            jax.ShapeDtypeStruct((P, DM // 2), jnp.int32),
            jax.ShapeDtypeStruct((P,), jnp.int32),
        ],
        mesh=mesh,
        scratch_types=[
            pltpu.VMEM((4, TPW), jnp.int32),
            pltpu.VMEM((TPW, DM // 2), jnp.int32),
            pltpu.SemaphoreType.DMA,
        ],
    )
    return f(x, meta_rs)


def _mm(a, b):
    return lax.dot_general(a, b, (((1,), (0,)), ((), ())),
                           preferred_element_type=jnp.float32)


def _gemm_kernel(blk_ref, xs_ref, ws_ref, wg_ref, wu_ref, wd_ref, y_ref):
    e = blk_ref[0, pl.program_id(0)]

    @pl.when(e < NE)
    def _body():
        xi = xs_ref[...]
        xlo = lax.bitcast_convert_type(
            xi << 16, jnp.float32).astype(jnp.bfloat16)
        xhi = lax.bitcast_convert_type(
            xi & jnp.int32(-65536), jnp.float32).astype(jnp.bfloat16)
        wge = wg_ref[e]
        wue = wu_ref[e]
        g = _mm(xlo, wge[:DM // 2]) + _mm(xhi, wge[DM // 2:])
        u = _mm(xlo, wue[:DM // 2]) + _mm(xhi, wue[DM // 2:])
        h = (g / (1.0 + jnp.exp(-g))) * u
        hb = (h * ws_ref[...]).astype(jnp.bfloat16)
        y_ref[...] = _mm(hb, wd_ref[e])


def _gemm(blk, xs_bits, wcol, wg, wu, wd):
    return pl.pallas_call(
        _gemm_kernel,
        grid=(NBLK,),
        in_specs=[
            pl.BlockSpec(memory_space=pltpu.MemorySpace.SMEM),
            pl.BlockSpec((BM2, DM // 2), lambda i: (i, 0)),
            pl.BlockSpec((BM2, 1), lambda i: (i, 0)),
            pl.BlockSpec((NE, DM, DF), lambda i: (0, 0, 0)),
            pl.BlockSpec((NE, DM, DF), lambda i: (0, 0, 0)),
            pl.BlockSpec((NE, DF, DM), lambda i: (0, 0, 0)),
        ],
        out_specs=pl.BlockSpec((BM2, DM), lambda i: (i, 0)),
        out_shape=jax.ShapeDtypeStruct((P, DM), jnp.float32),
    )(blk, xs_bits, wcol, wg, wu, wd)


def _combine_kernel(y_hbm, d0_hbm, d1_hbm, out_hbm,
                    idx0_v, idx1_v, buf0_v, buf1_v, sem):
    c = lax.axis_index("c")
    s = lax.axis_index("s")
    half = TPW // 2
    for h in range(2):
        row = (c * NTILE + s) * 2 + h
        pltpu.sync_copy(d0_hbm.at[row], idx0_v)
        pltpu.sync_copy(d1_hbm.at[row], idx1_v)
        cp0 = pltpu.make_async_copy(y_hbm.at[idx0_v], buf0_v, sem)
        cp0.start()
        cp1 = pltpu.make_async_copy(y_hbm.at[idx1_v], buf1_v, sem)
        cp1.start()
        cp0.wait()
        cp1.wait()

        def _add_row(r, _):
            for j in range(DM // 16):
                sl = pl.ds(j * 16, 16)
                buf0_v[r, sl] = buf0_v[r, sl] + buf1_v[r, sl]
            return 0

        lax.fori_loop(0, half, _add_row, 0)
        base = (c * NTILE + s) * TPW + h * half
        pltpu.sync_copy(buf0_v, out_hbm.at[pl.ds(base, half)])


def _combine(y, d0_rs, d1_rs):
    mesh = plsc.VectorSubcoreMesh(core_axis_name="c", subcore_axis_name="s")
    half = TPW // 2
    f = pl.kernel(
        _combine_kernel,
        out_type=jax.ShapeDtypeStruct((T, DM), jnp.float32),
        mesh=mesh,
        scratch_types=[
            pltpu.VMEM((half,), jnp.int32),
            pltpu.VMEM((half,), jnp.int32),
            pltpu.VMEM((half, DM), jnp.float32),
            pltpu.VMEM((half, DM), jnp.float32),
            pltpu.SemaphoreType.DMA,
        ],
    )
    return f(y, d0_rs, d1_rs)


@jax.jit
def kernel(hidden_states, proto, w_gate, w_up, w_down):
    B, S, D = hidden_states.shape
    x = hidden_states.reshape(T, D)
    protoT = proto.T
    wg = w_gate.astype(jnp.bfloat16)
    wu = w_up.astype(jnp.bfloat16)
    wd = w_down.astype(jnp.bfloat16)

    logits, meta, blk, x_bits = _route(x, protoT)

    meta_rs = meta.reshape(NW, TPW, 4).transpose(0, 2, 1)
    xs_bits, wsort = _dispatch(x_bits, meta_rs)

    wcol = lax.bitcast_convert_type(wsort, jnp.float32).reshape(P, 1)
    y = _gemm(blk, xs_bits, wcol, wg, wu, wd)

    d0_rs = meta[:, 0].reshape(64, 32)
    d1_rs = meta[:, 1].reshape(64, 32)
    out = _combine(y, d0_rs, d1_rs)
    return out.reshape(B, S, D), logits


# speedup vs baseline: 2.7867x; 1.2193x over previous
import functools

import jax
import jax.numpy as jnp
from jax import lax
from jax.experimental import pallas as pl
from jax.experimental.pallas import tpu as pltpu
from jax.experimental.pallas import tpu_sc as plsc

NE = 8
DM = 1024
DF = 512
T = 2048
BM2 = 256
NBLK = 24
P = NBLK * BM2
NSC = 2
NTILE = 16
NW = NSC * NTILE
TPW = T // NW


def _route_kernel(x_ref, protoT_ref, logits_ref, meta_ref, blk_ref,
                  xbits_ref, w0w_ref, w1w_ref):
    x = x_ref[...]
    xsq = jnp.sum(x * x, axis=1, keepdims=True)
    xn = x / jnp.maximum(jnp.sqrt(xsq), 1e-12)
    pT = protoT_ref[...]
    psq = jnp.sum(pT * pT, axis=0, keepdims=True)
    pn = pT / jnp.maximum(jnp.sqrt(psq), 1e-12)
    logits = lax.dot_general(
        xn.astype(jnp.bfloat16), pn.astype(jnp.bfloat16),
        (((1,), (0,)), ((), ())), preferred_element_type=jnp.float32)
    logits_ref[...] = logits

    m = jnp.max(logits, axis=1, keepdims=True)
    ex = jnp.exp(logits - m)
    probs = ex / jnp.sum(ex, axis=1, keepdims=True)
    p1 = jnp.max(probs, axis=1, keepdims=True)
    masked = jnp.where(probs >= p1, -jnp.inf, probs)
    p2 = jnp.max(masked, axis=1, keepdims=True)
    sel = probs >= p2

    mask_i = sel.astype(jnp.int32)
    c = mask_i
    s = 1
    while s < T:
        c = c + jnp.concatenate(
            [jnp.zeros((s, NE), jnp.int32), c[:T - s]], axis=0)
        s *= 2
    excl = c - mask_i

    lane1 = lax.broadcasted_iota(jnp.int32, (1, NE), 1)
    offs_row = jnp.zeros((1, NE), jnp.int32)
    blk = jnp.zeros((1, NBLK), jnp.int32)
    bstart = lax.broadcasted_iota(jnp.int32, (1, NBLK), 1) * BM2
    off_e = jnp.zeros((), jnp.int32)
    for e in range(NE):
        cnt_e = jnp.sum(mask_i[:, e:e + 1])
        cp_e = ((cnt_e + BM2 - 1) >> 8) << 8
        offs_row = offs_row + jnp.where(lane1 == e, off_e, 0)
        off_e = off_e + cp_e
        blk = blk + (bstart >= off_e).astype(jnp.int32)
    blk_ref[...] = blk

    dest = offs_row + excl
    one1 = probs >= p1
    one2 = sel & (~one1)
    d0 = jnp.sum(jnp.where(one1, dest, 0), axis=1, keepdims=True)
    d1 = jnp.sum(jnp.where(one2, dest, 0), axis=1, keepdims=True)
    w0 = lax.bitcast_convert_type(p1, jnp.int32)
    w1 = lax.bitcast_convert_type(p2, jnp.int32)
    meta = jnp.concatenate([d0, d1, w0, w1], axis=1)
    meta_ref[...] = meta
    w0w_ref[...] = jnp.broadcast_to(p1, (T, 16))
    w1w_ref[...] = jnp.broadcast_to(p2, (T, 16))

    lo = lax.shift_right_logical(
        lax.bitcast_convert_type(
            x[:, :DM // 2].astype(jnp.bfloat16).astype(jnp.float32),
            jnp.int32), 16)
    hi = lax.bitcast_convert_type(
        x[:, DM // 2:].astype(jnp.bfloat16).astype(jnp.float32),
        jnp.int32) & jnp.int32(-65536)
    xbits_ref[...] = lo | hi


def _route(x, protoT):
    return pl.pallas_call(
        _route_kernel,
        in_specs=[
            pl.BlockSpec((T, DM), lambda: (0, 0)),
            pl.BlockSpec((DM, NE), lambda: (0, 0)),
        ],
        out_specs=[
            pl.BlockSpec((T, NE), lambda: (0, 0)),
            pl.BlockSpec((T, 4), lambda: (0, 0)),
            pl.BlockSpec((1, NBLK), lambda: (0, 0)),
            pl.BlockSpec((T, DM // 2), lambda: (0, 0)),
            pl.BlockSpec((T, 16), lambda: (0, 0)),
            pl.BlockSpec((T, 16), lambda: (0, 0)),
        ],
        out_shape=[
            jax.ShapeDtypeStruct((T, NE), jnp.float32),
            jax.ShapeDtypeStruct((T, 4), jnp.int32),
            jax.ShapeDtypeStruct((1, NBLK), jnp.int32),
            jax.ShapeDtypeStruct((T, DM // 2), jnp.int32),
            jax.ShapeDtypeStruct((T, 16), jnp.float32),
            jax.ShapeDtypeStruct((T, 16), jnp.float32),
        ],
    )(x, protoT)


def _dispatch_kernel(x_hbm, meta_hbm, xs_hbm, meta_v, rows_v, sem):
    c = lax.axis_index("c")
    s = lax.axis_index("s")
    wid = s * NSC + c
    base = wid * TPW
    pltpu.sync_copy(meta_hbm.at[wid], meta_v)
    pltpu.sync_copy(x_hbm.at[pl.ds(base, TPW)], rows_v)
    cps = [
        pltpu.make_async_copy(rows_v, xs_hbm.at[meta_v.at[0]], sem),
        pltpu.make_async_copy(rows_v, xs_hbm.at[meta_v.at[1]], sem),
    ]
    for cp in cps:
        cp.start()
    for cp in cps:
        cp.wait()


def _dispatch(x, meta_rs):
    mesh = plsc.VectorSubcoreMesh(core_axis_name="c", subcore_axis_name="s")
    f = pl.kernel(
        _dispatch_kernel,
        out_type=jax.ShapeDtypeStruct((P, DM // 2), jnp.int32),
        mesh=mesh,
        scratch_types=[
            pltpu.VMEM((4, TPW), jnp.int32),
            pltpu.VMEM((TPW, DM // 2), jnp.int32),
            pltpu.SemaphoreType.DMA,
        ],
    )
    return f(x, meta_rs)


def _mm(a, b):
    return lax.dot_general(a, b, (((1,), (0,)), ((), ())),
                           preferred_element_type=jnp.float32)


def _gemm_kernel(blk_ref, xs_ref, wg_ref, wu_ref, wd_ref, y_ref):
    e = blk_ref[0, pl.program_id(0)]

    @pl.when(e < NE)
    def _body():
        xi = xs_ref[...]
        xlo = lax.bitcast_convert_type(
            xi << 16, jnp.float32).astype(jnp.bfloat16)
        xhi = lax.bitcast_convert_type(
            xi & jnp.int32(-65536), jnp.float32).astype(jnp.bfloat16)
        wge = wg_ref[e]
        wue = wu_ref[e]
        g = _mm(xlo, wge[:DM // 2]) + _mm(xhi, wge[DM // 2:])
        u = _mm(xlo, wue[:DM // 2]) + _mm(xhi, wue[DM // 2:])
        h = (g / (1.0 + jnp.exp(-g))) * u
        hb = h.astype(jnp.bfloat16)
        y_ref[...] = _mm(hb, wd_ref[e])


def _gemm(blk, xs_bits, wg, wu, wd):
    return pl.pallas_call(
        _gemm_kernel,
        grid=(NBLK,),
        in_specs=[
            pl.BlockSpec(memory_space=pltpu.MemorySpace.SMEM),
            pl.BlockSpec((BM2, DM // 2), lambda i: (i, 0)),
            pl.BlockSpec((NE, DM, DF), lambda i: (0, 0, 0)),
            pl.BlockSpec((NE, DM, DF), lambda i: (0, 0, 0)),
            pl.BlockSpec((NE, DF, DM), lambda i: (0, 0, 0)),
        ],
        out_specs=pl.BlockSpec((BM2, DM), lambda i: (i, 0)),
        out_shape=jax.ShapeDtypeStruct((P, DM), jnp.float32),
    )(blk, xs_bits, wg, wu, wd)


def _combine_kernel(y_hbm, d0_hbm, d1_hbm, w0_hbm, w1_hbm, out_hbm,
                    idx0_v, idx1_v, wv0_v, wv1_v, buf0_v, buf1_v, sem):
    c = lax.axis_index("c")
    s = lax.axis_index("s")
    half = TPW // 2
    for h in range(2):
        row = (c * NTILE + s) * 2 + h
        pltpu.sync_copy(d0_hbm.at[row], idx0_v)
        pltpu.sync_copy(d1_hbm.at[row], idx1_v)
        tbase = (c * NTILE + s) * TPW + h * half
        pltpu.sync_copy(w0_hbm.at[pl.ds(tbase, half)], wv0_v)
        pltpu.sync_copy(w1_hbm.at[pl.ds(tbase, half)], wv1_v)
        cp0 = pltpu.make_async_copy(y_hbm.at[idx0_v], buf0_v, sem)
        cp0.start()
        cp1 = pltpu.make_async_copy(y_hbm.at[idx1_v], buf1_v, sem)
        cp1.start()
        cp0.wait()
        cp1.wait()

        def _add_row(r, _):
            b0 = wv0_v[r, :]
            b1 = wv1_v[r, :]
            for j in range(DM // 16):
                sl = pl.ds(j * 16, 16)
                buf0_v[r, sl] = b0 * buf0_v[r, sl] + b1 * buf1_v[r, sl]
            return 0

        lax.fori_loop(0, half, _add_row, 0)
        base = (c * NTILE + s) * TPW + h * half
        pltpu.sync_copy(buf0_v, out_hbm.at[pl.ds(base, half)])


def _combine(y, d0_rs, d1_rs, w0_rs, w1_rs):
    mesh = plsc.VectorSubcoreMesh(core_axis_name="c", subcore_axis_name="s")
    half = TPW // 2
    f = pl.kernel(
        _combine_kernel,
        out_type=jax.ShapeDtypeStruct((T, DM), jnp.float32),
        mesh=mesh,
        scratch_types=[
            pltpu.VMEM((half,), jnp.int32),
            pltpu.VMEM((half,), jnp.int32),
            pltpu.VMEM((half, 16), jnp.float32),
            pltpu.VMEM((half, 16), jnp.float32),
            pltpu.VMEM((half, DM), jnp.float32),
            pltpu.VMEM((half, DM), jnp.float32),
            pltpu.SemaphoreType.DMA,
        ],
    )
    return f(y, d0_rs, d1_rs, w0_rs, w1_rs)


@jax.jit
def kernel(hidden_states, proto, w_gate, w_up, w_down):
    B, S, D = hidden_states.shape
    x = hidden_states.reshape(T, D)
    protoT = proto.T
    wg = w_gate.astype(jnp.bfloat16)
    wu = w_up.astype(jnp.bfloat16)
    wd = w_down.astype(jnp.bfloat16)

    logits, meta, blk, x_bits, w0w, w1w = _route(x, protoT)

    meta_rs = meta.reshape(NW, TPW, 4).transpose(0, 2, 1)
    xs_bits = _dispatch(x_bits, meta_rs)

    y = _gemm(blk, xs_bits, wg, wu, wd)

    d0_rs = meta[:, 0].reshape(64, 32)
    d1_rs = meta[:, 1].reshape(64, 32)
    out = _combine(y, d0_rs, d1_rs, w0w, w1w)
    return out.reshape(B, S, D), logits
